# baseline (speedup 1.0000x reference)
import functools

import jax
import jax.numpy as jnp
from jax.experimental import pallas as pl

N_I = 256
N_P = 8192
N_C = 9
LANES = 9 * 128
SUBS = N_P * N_C // LANES
I_BLK = 8


def _rot_stencil_kernel(c_ref, x_ref, o_ref):
    x = x_ref[...]
    c = c_ref[...]

    def tap(delta, idx):
        coef = c[:, idx, :][:, None, :]
        if delta == 0:
            return x * coef
        return jnp.roll(x, delta, axis=-1) * coef

    acc = tap(0, 2)
    acc = acc + tap(2, 0)
    acc = acc + tap(1, 1)
    acc = acc + tap(-1, 3)
    acc = acc + tap(-2, 4)
    o_ref[...] = acc


def _build_coefs(rot_mats):
    d = jnp.arange(LANES) % N_C
    cols = []
    for delta in (-2, -1, 0, 1, 2):
        c = d + delta
        valid = (d < 3) & (c >= 0) & (c < 3)
        dd = jnp.clip(d, 0, 2)
        cc = jnp.clip(c, 0, 2)
        vals = rot_mats[:, dd, cc]
        base = jnp.where((d >= 3) & (delta == 0), 1.0, 0.0)
        cols.append(jnp.where(valid[None, :], vals, base[None, :]))
    return jnp.stack(cols, axis=1)


@functools.partial(jax.jit, static_argnames=("interpret",))
def kernel(points_colored_instance, rot_mats, interpret=False):
    x = points_colored_instance.reshape(N_I, SUBS, LANES)
    coefs = _build_coefs(rot_mats)
    out = pl.pallas_call(
        _rot_stencil_kernel,
        grid=(N_I // I_BLK,),
        in_specs=[
            pl.BlockSpec((I_BLK, 5, LANES), lambda i: (i, 0, 0)),
            pl.BlockSpec((I_BLK, SUBS, LANES), lambda i: (i, 0, 0)),
        ],
        out_specs=pl.BlockSpec((I_BLK, SUBS, LANES), lambda i: (i, 0, 0)),
        out_shape=jax.ShapeDtypeStruct((N_I, SUBS, LANES), jnp.float32),
        interpret=interpret,
    )(coefs, x)
    return out.reshape(N_I, N_P, N_C)
